# baseline (speedup 1.0000x reference)
import functools

import jax
import jax.numpy as jnp
from jax import lax
from jax.experimental import pallas as pl
from jax.experimental.pallas import tpu as pltpu
from jax.experimental.pallas import tpu_sc as plsc

BATCH = 16384
LATENT_DIM = 32
HIDDEN = 16
LANES = 16

_info = plsc.get_sparse_core_info()
_NC, _NS = _info.num_cores, _info.num_subcores
_NW = _NC * _NS
_BPW = BATCH // _NW
_GROUPS = _BPW // LANES

_mesh = plsc.VectorSubcoreMesh(core_axis_name="c", subcore_axis_name="s")


@functools.partial(
    pl.kernel,
    mesh=_mesh,
    out_type=jax.ShapeDtypeStruct((BATCH,), jnp.float32),
    compiler_params=pltpu.CompilerParams(
        needs_layout_passes=False, use_tc_tiling_on_sc=False),
    scratch_types=[
        pltpu.VMEM((_BPW,), jnp.int32),
        pltpu.VMEM((_BPW,), jnp.int32),
        pltpu.VMEM((_BPW, LATENT_DIM), jnp.float32),
        pltpu.VMEM((_BPW, LATENT_DIM), jnp.float32),
        pltpu.VMEM((HIDDEN, LATENT_DIM), jnp.float32),
        pltpu.VMEM((HIDDEN,), jnp.float32),
        pltpu.SMEM((HIDDEN, LATENT_DIM), jnp.float32),
        pltpu.SMEM((HIDDEN,), jnp.float32),
        pltpu.VMEM((_BPW,), jnp.float32),
        pltpu.SemaphoreType.DMA,
        pltpu.SemaphoreType.DMA,
    ],
)
def _mf_sc(uids_hbm, iids_hbm, utab_hbm, itab_hbm, w_hbm, b_hbm, out_hbm,
           uidx_v, iidx_v, urows_v, irows_v, w_v, b_v, w_s, b_s, out_v,
           sem_u, sem_i):
    wid = lax.axis_index("s") * _NC + lax.axis_index("c")
    base = wid * _BPW

    pltpu.sync_copy(uids_hbm.at[pl.ds(base, _BPW)], uidx_v)
    pltpu.sync_copy(iids_hbm.at[pl.ds(base, _BPW)], iidx_v)
    cu = pltpu.async_copy(utab_hbm.at[uidx_v], urows_v, sem_u)
    ci = pltpu.async_copy(itab_hbm.at[iidx_v], irows_v, sem_i)
    pltpu.sync_copy(w_hbm, w_v)
    pltpu.sync_copy(b_hbm, b_v)

    for k in range(HIDDEN):
        lo = w_v[k, pl.ds(0, LANES)]
        hi = w_v[k, pl.ds(LANES, LANES)]
        for d in range(LANES):
            w_s[k, d] = lo[d]
            w_s[k, LANES + d] = hi[d]
    bvec = b_v[pl.ds(0, LANES)]
    for k in range(HIDDEN):
        b_s[k] = bvec[k]

    cu.wait()
    ci.wait()

    lane = lax.iota(jnp.int32, LANES)

    def group(g, _):
        row = g * LANES + lane
        p = []
        q = []
        for k in range(HIDDEN):
            bk = lax.broadcast(b_s[k], (LANES,))
            p.append(bk)
            q.append(bk)
        for d in range(LATENT_DIM):
            col = jnp.full((LANES,), d, dtype=jnp.int32)
            ud = plsc.load_gather(urows_v, [row, col])
            vd = plsc.load_gather(irows_v, [row, col])
            for k in range(HIDDEN):
                w = w_s[k, d]
                p[k] = p[k] + ud * w
                q[k] = q[k] + vd * w
        acc = p[0] * q[0]
        for k in range(1, HIDDEN):
            acc = acc + p[k] * q[k]
        out_v[pl.ds(g * LANES, LANES)] = acc
        return 0

    lax.fori_loop(0, _GROUPS, group, 0)
    pltpu.sync_copy(out_v, out_hbm.at[pl.ds(base, _BPW)])


def kernel(user_ids, item_ids, user_table, item_table, W_user, b_user):
    return _mf_sc(user_ids.astype(jnp.int32), item_ids.astype(jnp.int32),
                  user_table, item_table, W_user, b_user)
